# Initial kernel scaffold; baseline (speedup 1.0000x reference)
import functools

import jax
import jax.numpy as jnp
from jax import lax
from jax.experimental import pallas as pl
from jax.experimental.pallas import tpu as pltpu
from jax.experimental.pallas import tpu_sc as plsc

N = 100000
E = 1600000
IN_DIM = 128
HID_DIM = 128
OUT_DIM = 32
K = 10
ALPHA = 0.1

L = 16
NS = 16
NC = 2
ND = N // NS
NCH = 10
RCH = ND // NCH

CE = 2048
SB = 128
NSB = CE // SB
NCE = 50
ET = CE * NCE
EP = ET * NS
EPR = EP // SB
ERT = EPR // NS

_f32 = jnp.float32
_i32 = jnp.int32


def _mesh():
    return plsc.VectorSubcoreMesh(core_axis_name="c", subcore_axis_name="s")


def _deg_kernel(e2, deg, table, dst_buf, ones_buf, zero_buf, bounce, sem):
    cid = lax.axis_index("c")
    sid = lax.axis_index("s")
    nb0 = sid * ND

    for role in range(2):
        @pl.loop(0, NCH)
        def _zero(c):
            pltpu.sync_copy(zero_buf, table.at[pl.ds(nb0 + c * RCH, RCH), :])

        plsc.subcore_barrier()

        @pl.loop(0, NCE)
        def _scatter(c):
            rw = sid * ERT + c * NSB
            pltpu.sync_copy(e2.at[cid, role, pl.ds(rw, NSB), :], dst_buf)
            descs = [
                pltpu.async_copy(ones_buf, table.at[dst_buf.at[j]], sem,
                                 add=True)
                for j in range(NSB)
            ]
            for d in descs:
                d.wait()

        plsc.subcore_barrier()

        @pl.loop(0, NCH)
        def _drain(c):
            nb = nb0 + c * RCH
            pltpu.sync_copy(table.at[pl.ds(nb, RCH), :], bounce)
            pltpu.sync_copy(bounce, deg.at[cid, role, pl.ds(nb, RCH), :])

        plsc.subcore_barrier()


def _run_deg(e2):
    kfn = pl.kernel(
        _deg_kernel,
        out_type=jax.ShapeDtypeStruct((2, 2, N, L), _f32),
        mesh=_mesh(),
        scratch_types=[
            pltpu.VMEM_SHARED((N + 8, L), _f32),
            pltpu.VMEM((NSB, SB), _i32),
            pltpu.VMEM((SB, L), _f32),
            pltpu.VMEM((RCH, L), _f32),
            pltpu.VMEM((RCH, L), _f32),
            pltpu.SemaphoreType.DMA,
        ],
    )
    return kfn(e2)


def _mlp_kernel(x, w1, b1, w2, b2, deg, p0, c1, c3):
    h = jnp.dot(x[...], w1[...], preferred_element_type=_f32) + b1[...]
    h = jax.nn.relu(h)
    h0 = jnp.dot(h, w2[...], preferred_element_type=_f32) + b2[...]
    for v in range(2):
        ns = lax.rsqrt(jnp.maximum(deg[v, 0], 1.0))
        nd = lax.rsqrt(jnp.maximum(deg[v, 1], 1.0))
        c1[v] = (1.0 - ALPHA) * nd * ns
        c3[v] = 0.5 / ns
        for hh in range(2):
            p0[v, hh] = h0[:, hh * L:(hh + 1) * L] * ns


def _run_mlp(X, W1, b1, W2, b2, deg):
    RB = 1000
    grid = (N // RB,)
    return pl.pallas_call(
        _mlp_kernel,
        grid=grid,
        in_specs=[
            pl.BlockSpec((RB, IN_DIM), lambda i: (i, 0)),
            pl.BlockSpec((IN_DIM, HID_DIM), lambda i: (0, 0)),
            pl.BlockSpec((1, HID_DIM), lambda i: (0, 0)),
            pl.BlockSpec((HID_DIM, OUT_DIM), lambda i: (0, 0)),
            pl.BlockSpec((1, OUT_DIM), lambda i: (0, 0)),
            pl.BlockSpec((2, 2, RB, L), lambda i: (0, 0, i, 0)),
        ],
        out_specs=[
            pl.BlockSpec((2, 2, RB, L), lambda i: (0, 0, i, 0)),
            pl.BlockSpec((2, RB, L), lambda i: (0, i, 0)),
            pl.BlockSpec((2, RB, L), lambda i: (0, i, 0)),
        ],
        out_shape=[
            jax.ShapeDtypeStruct((2, 2, N, L), _f32),
            jax.ShapeDtypeStruct((2, N, L), _f32),
            jax.ShapeDtypeStruct((2, N, L), _f32),
        ],
    )(X, W1, b1.reshape(1, -1), W2, b2.reshape(1, -1), deg)


def _prop_kernel(e1, e2, p0, c1, c3, out, pw,
                 agg_sh, src_buf, dst_buf, rows_buf,
                 zero_buf, av, bv, cv, ov, sem_g, sem_s):
    cid = lax.axis_index("c")
    sid = lax.axis_index("s")
    nb0 = sid * ND

    for v in range(2):
        vrow = (2 * v + cid) * N

        @pl.loop(0, NCH)
        def _init(c):
            nb = nb0 + c * RCH
            pltpu.sync_copy(p0.at[pl.ds(vrow + nb, RCH), :], bv)
            pltpu.sync_copy(bv, pw.at[pl.ds(vrow + nb, RCH), :])

        plsc.subcore_barrier()

        @pl.loop(0, K)
        def _round(k):
            @pl.loop(0, NCH)
            def _zero(c):
                pltpu.sync_copy(zero_buf,
                                agg_sh.at[pl.ds(nb0 + c * RCH, RCH), :])

            plsc.subcore_barrier()

            @pl.loop(0, NCE)
            def _edges(c):
                off = sid * ET + c * CE
                rw = sid * ERT + c * NSB
                pltpu.sync_copy(e1.at[v, 0, pl.ds(off, CE)], src_buf)
                pltpu.sync_copy(e2.at[v, 1, pl.ds(rw, NSB), :], dst_buf)

                @pl.loop(0, CE // L)
                def _base(i):
                    sl = pl.ds(i * L, L)
                    src_buf[sl] = src_buf[sl] + vrow

                gd = [
                    pltpu.async_copy(
                        pw.at[src_buf.at[pl.ds(j * SB, SB)]],
                        rows_buf.at[pl.ds(j * SB, SB), :], sem_g)
                    for j in range(NSB)
                ]
                for d in gd:
                    d.wait()
                sd = [
                    pltpu.async_copy(
                        rows_buf.at[pl.ds(j * SB, SB), :],
                        agg_sh.at[dst_buf.at[j]], sem_s, add=True)
                    for j in range(NSB)
                ]
                for d in sd:
                    d.wait()

            plsc.subcore_barrier()

            @pl.loop(0, NCH)
            def _drain(c):
                nb = nb0 + c * RCH
                pltpu.sync_copy(agg_sh.at[pl.ds(nb, RCH), :], av)
                pltpu.sync_copy(p0.at[pl.ds(vrow + nb, RCH), :], bv)
                pltpu.sync_copy(c1.at[pl.ds(v * N + nb, RCH), :], cv)

                @pl.loop(0, RCH, unroll=4)
                def _rows(r):
                    av[r] = cv[r] * av[r] + ALPHA * bv[r]

                pltpu.sync_copy(av, pw.at[pl.ds(vrow + nb, RCH), :])

            plsc.subcore_barrier()

    @pl.loop(0, NCH)
    def _fin(c):
        nb = nb0 + c * RCH
        pltpu.sync_copy(pw.at[pl.ds(cid * N + nb, RCH), :], av)
        pltpu.sync_copy(c3.at[pl.ds(nb, RCH), :], cv)

        @pl.loop(0, RCH, unroll=4)
        def _rows0(r):
            ov[r] = av[r] * cv[r]

        pltpu.sync_copy(pw.at[pl.ds((2 + cid) * N + nb, RCH), :], av)
        pltpu.sync_copy(c3.at[pl.ds(N + nb, RCH), :], cv)

        @pl.loop(0, RCH, unroll=4)
        def _rows1(r):
            ov[r] = ov[r] + av[r] * cv[r]

        pltpu.sync_copy(ov, out.at[pl.ds(cid * N + nb, RCH), :])


def _run_prop(e1, e2, p0f, c1f, c3f):
    kfn = pl.kernel(
        _prop_kernel,
        out_type=(
            jax.ShapeDtypeStruct((2 * N, L), _f32),
            jax.ShapeDtypeStruct((4 * N + 8, L), _f32),
        ),
        mesh=_mesh(),
        scratch_types=[
            pltpu.VMEM_SHARED((N + 8, L), _f32),
            pltpu.VMEM((CE,), _i32),
            pltpu.VMEM((NSB, SB), _i32),
            pltpu.VMEM((CE, L), _f32),
            pltpu.VMEM((RCH, L), _f32),
            pltpu.VMEM((RCH, L), _f32),
            pltpu.VMEM((RCH, L), _f32),
            pltpu.VMEM((RCH, L), _f32),
            pltpu.VMEM((RCH, L), _f32),
            pltpu.SemaphoreType.DMA,
            pltpu.SemaphoreType.DMA,
        ],
    )
    return kfn(e1, e2, p0f, c1f, c3f)


@jax.jit
def kernel(X, edge_index_v1, edge_index_v2, W1, b1, W2, b2):
    e_all = jnp.stack([edge_index_v1, edge_index_v2])
    pad = jnp.full((2, 2, EP - E), N, dtype=_i32)
    e1 = jnp.concatenate([e_all, pad], axis=2)
    e2 = e1.reshape(2, 2, EPR, SB)

    deg = _run_deg(e2)
    p0, c1, c3 = _run_mlp(X, W1, b1, W2, b2, deg)

    out2, _ = _run_prop(e1, e2, p0.reshape(4 * N, L),
                        c1.reshape(2 * N, L), c3.reshape(2 * N, L))
    return out2.reshape(2, N, L).transpose(1, 0, 2).reshape(N, OUT_DIM)


# speedup vs baseline: 7.1333x; 7.1333x over previous
import functools

import jax
import jax.numpy as jnp
from jax import lax
from jax.experimental import pallas as pl
from jax.experimental.pallas import tpu as pltpu
from jax.experimental.pallas import tpu_sc as plsc

N = 100000
E = 1600000
IN_DIM = 128
HID_DIM = 128
OUT_DIM = 32
K = 10
ALPHA = 0.1

L = 16
NS = 16
NC = 2
NP = 102400
ND = NP // NS
NCH = 40
RCH = ND // NCH

CE = 512
SB = 128
NSB = CE // SB
NCE = 200
ET = CE * NCE
EP = ET * NS
EPR = EP // SB
ERT = EPR // NS

_f32 = jnp.float32
_i32 = jnp.int32


def _mesh():
    return plsc.VectorSubcoreMesh(core_axis_name="c", subcore_axis_name="s")


def _deg_kernel(e2, deg, table, dst_buf, ones_buf, zero_buf, bounce, sem):
    cid = lax.axis_index("c")
    sid = lax.axis_index("s")
    nb0 = sid * ND

    @pl.loop(0, SB)
    def _ones(r):
        ones_buf[r] = jnp.full((L,), 1.0, _f32)

    @pl.loop(0, RCH)
    def _zinit(r):
        zero_buf[r] = jnp.zeros((L,), _f32)

    for role in range(2):
        @pl.loop(0, NCH)
        def _zero(c):
            pltpu.sync_copy(zero_buf, table.at[pl.ds(nb0 + c * RCH, RCH), :])

        plsc.subcore_barrier()

        @pl.loop(0, NCE)
        def _scatter(c):
            rw = sid * ERT + c * NSB
            pltpu.sync_copy(e2.at[cid, role, pl.ds(rw, NSB), :], dst_buf)
            descs = [
                pltpu.async_copy(ones_buf, table.at[dst_buf.at[j]], sem,
                                 add=True)
                for j in range(NSB)
            ]
            for d in descs:
                d.wait()

        plsc.subcore_barrier()

        @pl.loop(0, NCH)
        def _drain(c):
            nb = nb0 + c * RCH
            pltpu.sync_copy(table.at[pl.ds(nb, RCH), :], bounce)
            pltpu.sync_copy(bounce, deg.at[cid, role, pl.ds(nb, RCH), :])

        plsc.subcore_barrier()


def _run_deg(e2):
    kfn = pl.kernel(
        _deg_kernel,
        out_type=jax.ShapeDtypeStruct((2, 2, NP, L), _f32),
        mesh=_mesh(),
        compiler_params=pltpu.CompilerParams(use_tc_tiling_on_sc=False),
        scratch_types=[
            pltpu.VMEM_SHARED((NP, L), _f32),
            pltpu.VMEM((NSB, SB), _i32),
            pltpu.VMEM((SB, L), _f32),
            pltpu.VMEM((RCH, L), _f32),
            pltpu.VMEM((RCH, L), _f32),
            pltpu.SemaphoreType.DMA,
        ],
    )
    return kfn(e2)


def _mlp_kernel(x, w1, b1, w2, b2, deg, p0, c1, c3):
    h = jnp.dot(x[...], w1[...], preferred_element_type=_f32) + b1[...]
    h = jax.nn.relu(h)
    h0 = jnp.dot(h, w2[...], preferred_element_type=_f32) + b2[...]
    for v in range(2):
        ns = lax.rsqrt(jnp.maximum(deg[v, 0], 1.0))
        nd = lax.rsqrt(jnp.maximum(deg[v, 1], 1.0))
        c1[v] = (1.0 - ALPHA) * nd * ns
        c3[v] = 0.5 / ns
        for hh in range(2):
            p0[v, hh] = h0[:, hh * L:(hh + 1) * L] * ns


def _run_mlp(X, W1, b1, W2, b2, deg):
    RB = 1024
    grid = (NP // RB,)
    return pl.pallas_call(
        _mlp_kernel,
        grid=grid,
        in_specs=[
            pl.BlockSpec((RB, IN_DIM), lambda i: (i, 0)),
            pl.BlockSpec((IN_DIM, HID_DIM), lambda i: (0, 0)),
            pl.BlockSpec((1, HID_DIM), lambda i: (0, 0)),
            pl.BlockSpec((HID_DIM, OUT_DIM), lambda i: (0, 0)),
            pl.BlockSpec((1, OUT_DIM), lambda i: (0, 0)),
            pl.BlockSpec((2, 2, RB, L), lambda i: (0, 0, i, 0)),
        ],
        out_specs=[
            pl.BlockSpec((2, 2, RB, L), lambda i: (0, 0, i, 0)),
            pl.BlockSpec((2, RB, L), lambda i: (0, i, 0)),
            pl.BlockSpec((2, RB, L), lambda i: (0, i, 0)),
        ],
        out_shape=[
            jax.ShapeDtypeStruct((2, 2, NP, L), _f32),
            jax.ShapeDtypeStruct((2, NP, L), _f32),
            jax.ShapeDtypeStruct((2, NP, L), _f32),
        ],
    )(X, W1, b1.reshape(1, -1), W2, b2.reshape(1, -1), deg)


def _prop_kernel(e2, p0, c1, c3, out, pw,
                 agg_sh, src_buf, dst_buf, rows_buf,
                 zero_buf, av, bv, cv, sem_g, sem_s):
    cid = lax.axis_index("c")
    sid = lax.axis_index("s")
    nb0 = sid * ND

    @pl.loop(0, RCH)
    def _zinit(r):
        zero_buf[r] = jnp.zeros((L,), _f32)

    for v in range(2):
        vrow = (2 * v + cid) * NP

        @pl.loop(0, NCH)
        def _init(c):
            nb = nb0 + c * RCH
            pltpu.sync_copy(p0.at[pl.ds(vrow + nb, RCH), :], bv)
            pltpu.sync_copy(bv, pw.at[pl.ds(vrow + nb, RCH), :])

        plsc.subcore_barrier()

        @pl.loop(0, K)
        def _round(k):
            @pl.loop(0, NCH)
            def _zero(c):
                pltpu.sync_copy(zero_buf,
                                agg_sh.at[pl.ds(nb0 + c * RCH, RCH), :])

            plsc.subcore_barrier()

            @pl.loop(0, NCE)
            def _edges(c):
                rw = sid * ERT + c * NSB
                pltpu.sync_copy(e2.at[v, 0, pl.ds(rw, NSB), :], src_buf)
                pltpu.sync_copy(e2.at[v, 1, pl.ds(rw, NSB), :], dst_buf)

                @pl.loop(0, NSB)
                def _base(j):
                    for i in range(SB // L):
                        sl = pl.ds(i * L, L)
                        src_buf[j, sl] = src_buf[j, sl] + vrow

                gd = [
                    pltpu.async_copy(
                        pw.at[src_buf.at[j]],
                        rows_buf.at[pl.ds(j * SB, SB), :], sem_g)
                    for j in range(NSB)
                ]
                for d in gd:
                    d.wait()
                sd = [
                    pltpu.async_copy(
                        rows_buf.at[pl.ds(j * SB, SB), :],
                        agg_sh.at[dst_buf.at[j]], sem_s, add=True)
                    for j in range(NSB)
                ]
                for d in sd:
                    d.wait()

            plsc.subcore_barrier()

            @pl.loop(0, NCH)
            def _drain(c):
                nb = nb0 + c * RCH
                pltpu.sync_copy(agg_sh.at[pl.ds(nb, RCH), :], av)
                pltpu.sync_copy(p0.at[pl.ds(vrow + nb, RCH), :], bv)
                pltpu.sync_copy(c1.at[pl.ds(v * NP + nb, RCH), :], cv)

                @pl.loop(0, RCH, unroll=4)
                def _rows(r):
                    av[r] = cv[r] * av[r] + ALPHA * bv[r]

                pltpu.sync_copy(av, pw.at[pl.ds(vrow + nb, RCH), :])

            plsc.subcore_barrier()

    @pl.loop(0, NCH)
    def _fin(c):
        nb = nb0 + c * RCH
        pltpu.sync_copy(pw.at[pl.ds(cid * NP + nb, RCH), :], av)
        pltpu.sync_copy(c3.at[pl.ds(nb, RCH), :], cv)

        @pl.loop(0, RCH, unroll=4)
        def _rows0(r):
            bv[r] = av[r] * cv[r]

        pltpu.sync_copy(pw.at[pl.ds((2 + cid) * NP + nb, RCH), :], av)
        pltpu.sync_copy(c3.at[pl.ds(NP + nb, RCH), :], cv)

        @pl.loop(0, RCH, unroll=4)
        def _rows1(r):
            bv[r] = bv[r] + av[r] * cv[r]

        pltpu.sync_copy(bv, out.at[pl.ds(cid * NP + nb, RCH), :])


def _run_prop(e2, p0f, c1f, c3f):
    kfn = pl.kernel(
        _prop_kernel,
        out_type=(
            jax.ShapeDtypeStruct((2 * NP, L), _f32),
            jax.ShapeDtypeStruct((4 * NP, L), _f32),
        ),
        mesh=_mesh(),
        compiler_params=pltpu.CompilerParams(use_tc_tiling_on_sc=False),
        scratch_types=[
            pltpu.VMEM_SHARED((NP, L), _f32),
            pltpu.VMEM((NSB, SB), _i32),
            pltpu.VMEM((NSB, SB), _i32),
            pltpu.VMEM((CE, L), _f32),
            pltpu.VMEM((RCH, L), _f32),
            pltpu.VMEM((RCH, L), _f32),
            pltpu.VMEM((RCH, L), _f32),
            pltpu.VMEM((RCH, L), _f32),
            pltpu.SemaphoreType.DMA,
            pltpu.SemaphoreType.DMA,
        ],
    )
    return kfn(e2, p0f, c1f, c3f)


@jax.jit
def kernel(X, edge_index_v1, edge_index_v2, W1, b1, W2, b2):
    e_all = jnp.stack([edge_index_v1, edge_index_v2])
    pad = jnp.full((2, 2, EP - E), N, dtype=_i32)
    e1 = jnp.concatenate([e_all, pad], axis=2)
    e2 = e1.reshape(2, 2, EPR, SB)

    Xp = jnp.pad(X, ((0, NP - N), (0, 0)))

    deg = _run_deg(e2)
    p0, c1, c3 = _run_mlp(Xp, W1, b1, W2, b2, deg)

    out2, _ = _run_prop(e2, p0.reshape(4 * NP, L),
                        c1.reshape(2 * NP, L), c3.reshape(2 * NP, L))
    return (out2.reshape(2, NP, L)[:, :N]
            .transpose(1, 0, 2).reshape(N, OUT_DIM))
